# baseline (speedup 1.0000x reference)
import functools

import jax
import jax.numpy as jnp
from jax import lax
from jax.experimental import pallas as pl
from jax.experimental.pallas import tpu as pltpu
from jax.experimental.pallas import tpu_sc as plsc

NC = 2
NS = 16
NW = NC * NS
LANES = 16
EPW = 10368
E_PAD = NW * EPW


def _make_propagate(n_pad, d, C, SCK, nsc):
    rps = n_pad // NS
    nz = rps // C
    cg_n = d // LANES

    mesh = plsc.VectorSubcoreMesh(
        core_axis_name="c", subcore_axis_name="s", num_cores=NC, num_subcores=NS
    )

    @functools.partial(
        pl.kernel,
        out_type=jax.ShapeDtypeStruct((NC, n_pad, d), jnp.float32),
        mesh=mesh,
        compiler_params=pltpu.CompilerParams(use_tc_tiling_on_sc=(d >= 128)),
        scratch_types=[
            pltpu.VMEM((2, SCK, C), jnp.int32),
            pltpu.VMEM((SCK, C), jnp.float32),
            pltpu.VMEM((C, d), jnp.float32),
            pltpu.VMEM((C, d), jnp.float32),
            pltpu.VMEM((C, d), jnp.float32),
            pltpu.VMEM_SHARED((n_pad, d), jnp.float32),
            pltpu.SemaphoreType.DMA((3,)),
            pltpu.SemaphoreType.DMA((3,)),
        ],
    )
    def prop(z_hbm, e2_hbm, w_hbm, out_hbm, e3_v, w_v, r0, r1, r2, acc,
             gsem, ssem):
        bufs = (r0, r1, r2)
        cid = lax.axis_index("c")
        sid = lax.axis_index("s")
        wid = cid * NS + sid

        zeros16 = jnp.zeros((LANES,), jnp.float32)

        def zrow(r, carry):
            for cg in range(cg_n):
                r0[r, pl.ds(cg * LANES, LANES)] = zeros16
            return carry

        lax.fori_loop(0, C, zrow, 0)
        base = sid * rps
        for zi in range(nz):
            pltpu.sync_copy(r0, acc.at[pl.ds(base + zi * C, C)])
        plsc.subcore_barrier()

        def fire_gather(k, b):
            return pltpu.async_copy(
                z_hbm.at[e3_v.at[0, k]], bufs[b], gsem.at[b])

        def drain_scatter(k, b):
            pltpu.make_async_copy(
                bufs[b], acc.at[e3_v.at[1, k]], ssem.at[b]).wait()

        def scale(k, b):
            buf = bufs[b]
            for g in range(C // LANES):
                wg = w_v[k, pl.ds(g * LANES, LANES)]
                for i in range(LANES):
                    ee = g * LANES + i
                    wb = wg.at[jnp.full((LANES,), i, jnp.int32)].get(
                        mode="promise_in_bounds")
                    for cg in range(cg_n):
                        sl = pl.ds(cg * LANES, LANES)
                        buf[ee, sl] = buf[ee, sl] * wb

        def superchunk(j, carry):
            pltpu.sync_copy(e2_hbm.at[wid].at[j], e3_v)
            pltpu.sync_copy(w_hbm.at[wid].at[j], w_v)
            fire_gather(0, 0)
            fire_gather(1, 1)

            def triple(t, carry2):
                for b in range(3):
                    k = t * 3 + b
                    pltpu.make_async_copy(
                        z_hbm.at[e3_v.at[0, k]], bufs[b], gsem.at[b]).wait()
                    scale(k, b)
                    pltpu.async_copy(
                        bufs[b], acc.at[e3_v.at[1, k]], ssem.at[b], add=True)
                    nb = (b + 2) % 3
                    km1 = jnp.maximum(k - 1, 0)

                    @pl.when(k >= 1)
                    def _():
                        drain_scatter(km1, nb)

                    kp2 = jnp.minimum(k + 2, SCK - 1)

                    @pl.when(k + 2 <= SCK - 1)
                    def _():
                        fire_gather(kp2, nb)
                return carry2

            lax.fori_loop(0, SCK // 3, triple, 0)
            drain_scatter(SCK - 1, (SCK - 1) % 3)
            return carry

        lax.fori_loop(0, nsc, superchunk, 0)
        plsc.subcore_barrier()

        pltpu.sync_copy(acc.at[pl.ds(base, rps)],
                        out_hbm.at[cid].at[pl.ds(base, rps)])

    return prop


def _matmul(x, w):
    n, din = x.shape
    dout = w.shape[1]
    bm = 1000

    def body(x_ref, w_ref, o_ref):
        o_ref[...] = jnp.dot(x_ref[...], w_ref[...],
                             preferred_element_type=jnp.float32)

    return pl.pallas_call(
        body,
        grid=(n // bm,),
        in_specs=[pl.BlockSpec((bm, din), lambda i: (i, 0)),
                  pl.BlockSpec((din, dout), lambda i: (0, 0))],
        out_specs=pl.BlockSpec((bm, dout), lambda i: (i, 0)),
        out_shape=jax.ShapeDtypeStruct((n, dout), jnp.float32),
    )(x, w)


def _combine_relu_matmul(p, b1, w2, n):
    din = p.shape[2]
    dout = w2.shape[1]
    bm = 1000
    b1r = b1.reshape(1, din)

    def body(p_ref, b_ref, w_ref, o_ref):
        h = jnp.maximum(p_ref[0] + p_ref[1] + b_ref[...], 0.0)
        o_ref[...] = jnp.dot(h, w_ref[...], preferred_element_type=jnp.float32)

    return pl.pallas_call(
        body,
        grid=(n // bm,),
        in_specs=[pl.BlockSpec((2, bm, din), lambda i: (0, i, 0)),
                  pl.BlockSpec((1, din), lambda i: (0, 0)),
                  pl.BlockSpec((din, dout), lambda i: (0, 0))],
        out_specs=pl.BlockSpec((bm, dout), lambda i: (i, 0)),
        out_shape=jax.ShapeDtypeStruct((n, dout), jnp.float32),
    )(p, b1r, w2)


def _combine_bias(q, b2, n):
    d = q.shape[2]
    bm = 1000
    b2r = b2.reshape(1, d)

    def body(q_ref, b_ref, o_ref):
        o_ref[...] = q_ref[0] + q_ref[1] + b_ref[...]

    return pl.pallas_call(
        body,
        grid=(n // bm,),
        in_specs=[pl.BlockSpec((2, bm, d), lambda i: (0, i, 0)),
                  pl.BlockSpec((1, d), lambda i: (0, 0))],
        out_specs=pl.BlockSpec((bm, d), lambda i: (i, 0)),
        out_shape=jax.ShapeDtypeStruct((n, d), jnp.float32),
    )(q, b2r)


def _pack_edges(edge_index, edge_weight, n_pad, C, SCK, nsc):
    e = edge_index.shape[1]
    pad = E_PAD - e
    src = jnp.concatenate([edge_index[0], jnp.zeros((pad,), jnp.int32)])
    dst = jnp.concatenate(
        [edge_index[1], (jnp.arange(pad, dtype=jnp.int32) % n_pad)])
    wts = jnp.concatenate([edge_weight, jnp.zeros((pad,), jnp.float32)])
    parts = [a.reshape(NW, nsc, 1, SCK, C) for a in (src, dst)]
    e2 = jnp.concatenate(parts, axis=2)
    return e2, wts.reshape(NW, nsc, SCK, C)


def kernel(x, label, mask, edge_index, edge_weight, W1, b1, W2, b2):
    n, d_in = x.shape
    d_h = W1.shape[1]
    d_out = W2.shape[1]
    n_pad = 10240

    e2_l1, w_l1 = _pack_edges(edge_index, edge_weight, n_pad,
                              C=64, SCK=18, nsc=9)
    e2_l2, w_l2 = _pack_edges(edge_index, edge_weight, n_pad,
                              C=128, SCK=81, nsc=1)

    h0 = _matmul(x, W1)
    p1 = _make_propagate(n_pad, d_h, 64, 18, 9)(h0, e2_l1, w_l1)
    h1 = _combine_relu_matmul(p1, b1, W2, n)
    p2 = _make_propagate(n_pad, d_out, 128, 81, 1)(h1, e2_l2, w_l2)
    return _combine_bias(p2, b2, n)
